# Initial kernel scaffold; baseline (speedup 1.0000x reference)
import functools

import jax
import jax.numpy as jnp
from jax import lax
from jax.experimental import pallas as pl
from jax.experimental.pallas import tpu as pltpu
from jax.experimental.pallas import tpu_sc as plsc

N = 10000
E = 320000
NPAD = 10240
F1 = 128
F2 = 64
NC = 2
NS = 16
NW = NC * NS
EW = E // NW
CH = 80
NCH = EW // CH
NR = NPAD // NS


def _seg_sum_body(D, with_cnt, *refs):
    if with_cnt:
        (src2, dst2, x, zrows, zn, ones_c, s_out, cnt_out,
         src_v, dst_v, rows_v, ones_v, acc_sh, cnt_sh, sem) = refs
    else:
        (src2, dst2, x, zrows, s_out,
         src_v, dst_v, rows_v, acc_sh, sem) = refs
    cid = lax.axis_index("c")
    sid = lax.axis_index("s")
    wid = cid * NS + sid

    pltpu.sync_copy(zrows.at[pl.ds(sid * NR, NR)],
                    acc_sh.at[pl.ds(sid * NR, NR)])
    if with_cnt:
        pltpu.sync_copy(zn.at[pl.ds(sid * NR, NR)],
                        cnt_sh.at[pl.ds(sid * NR, NR)])
        pltpu.sync_copy(ones_c, ones_v)
    pltpu.sync_copy(src2.at[pl.ds(wid * NCH, NCH)], src_v)
    pltpu.sync_copy(dst2.at[pl.ds(wid * NCH, NCH)], dst_v)
    plsc.subcore_barrier()

    def chunk(j, carry):
        pltpu.async_copy(x.at[src_v.at[j]], rows_v, sem).wait()
        pltpu.sync_copy(rows_v, acc_sh.at[dst_v.at[j]], add=True)
        if with_cnt:
            pltpu.sync_copy(ones_v, cnt_sh.at[dst_v.at[j]], add=True)
        return carry

    lax.fori_loop(0, NCH, chunk, 0)
    plsc.subcore_barrier()

    pltpu.sync_copy(acc_sh.at[pl.ds(sid * NR, NR)],
                    s_out.at[cid, pl.ds(sid * NR, NR)])
    if with_cnt:
        pltpu.sync_copy(cnt_sh.at[pl.ds(sid * NR, NR)],
                        cnt_out.at[cid, pl.ds(sid * NR, NR)])


def _make_seg_sum(D, with_cnt):
    mesh = plsc.VectorSubcoreMesh(core_axis_name="c", subcore_axis_name="s",
                                  num_cores=NC, num_subcores=NS)
    out_type = [jax.ShapeDtypeStruct((NC, NPAD, D), jnp.float32)]
    scratch = [
        pltpu.VMEM((NCH, CH), jnp.int32),
        pltpu.VMEM((NCH, CH), jnp.int32),
        pltpu.VMEM((CH, D), jnp.float32),
    ]
    if with_cnt:
        out_type.append(jax.ShapeDtypeStruct((NC, NPAD), jnp.float32))
        scratch.append(pltpu.VMEM((CH,), jnp.float32))
    scratch.append(pltpu.VMEM_SHARED((NPAD, D), jnp.float32))
    if with_cnt:
        scratch.append(pltpu.VMEM_SHARED((NPAD,), jnp.float32))
    scratch.append(pltpu.SemaphoreType.DMA)
    return pl.kernel(functools.partial(_seg_sum_body, D, with_cnt),
                     out_type=out_type, mesh=mesh, scratch_types=scratch)


def _layer1_body(s_ref, c_ref, x_ref, a1_ref, b1_ref, r1_ref, c2_ref,
                 h_ref, y2_ref):
    s = s_ref[0] + s_ref[1]
    cnt = c_ref[0] + c_ref[1]
    mean = s / jnp.maximum(cnt, 1.0)
    h = (jnp.dot(mean, a1_ref[...], preferred_element_type=jnp.float32)
         + b1_ref[...]
         + jnp.dot(x_ref[...], r1_ref[...], preferred_element_type=jnp.float32))
    h_ref[...] = h
    y2_ref[...] = jnp.dot(h, c2_ref[...], preferred_element_type=jnp.float32)


def _layer2_body(s_ref, c_ref, h_ref, r2_ref, b2_ref, o_ref):
    s = s_ref[0] + s_ref[1]
    cnt = c_ref[0] + c_ref[1]
    mean = s / jnp.maximum(cnt, 1.0)
    o_ref[...] = (mean + b2_ref[...]
                  + jnp.dot(h_ref[...], r2_ref[...],
                            preferred_element_type=jnp.float32))


BR = 1024
GRID = NPAD // BR


def _tc_layer1(s1, cntr, x, a1, b1, r1, c2):
    return pl.pallas_call(
        _layer1_body,
        grid=(GRID,),
        in_specs=[
            pl.BlockSpec((NC, BR, F1), lambda i: (0, i, 0)),
            pl.BlockSpec((NC, BR, 1), lambda i: (0, i, 0)),
            pl.BlockSpec((BR, F1), lambda i: (i, 0)),
            pl.BlockSpec((F1, F1), lambda i: (0, 0)),
            pl.BlockSpec((1, F1), lambda i: (0, 0)),
            pl.BlockSpec((F1, F1), lambda i: (0, 0)),
            pl.BlockSpec((F1, F2), lambda i: (0, 0)),
        ],
        out_specs=[
            pl.BlockSpec((BR, F1), lambda i: (i, 0)),
            pl.BlockSpec((BR, F2), lambda i: (i, 0)),
        ],
        out_shape=[
            jax.ShapeDtypeStruct((NPAD, F1), jnp.float32),
            jax.ShapeDtypeStruct((NPAD, F2), jnp.float32),
        ],
    )(s1, cntr, x, a1, b1, r1, c2)


def _tc_layer2(s2, cntr, h, r2, b2):
    return pl.pallas_call(
        _layer2_body,
        grid=(GRID,),
        in_specs=[
            pl.BlockSpec((NC, BR, F2), lambda i: (0, i, 0)),
            pl.BlockSpec((NC, BR, 1), lambda i: (0, i, 0)),
            pl.BlockSpec((BR, F1), lambda i: (i, 0)),
            pl.BlockSpec((F1, F2), lambda i: (0, 0)),
            pl.BlockSpec((1, F2), lambda i: (0, 0)),
        ],
        out_specs=pl.BlockSpec((BR, F2), lambda i: (i, 0)),
        out_shape=jax.ShapeDtypeStruct((NPAD, F2), jnp.float32),
    )(s2, cntr, h, r2, b2)


def kernel(g, embeds, W_l1, b_l1, W_r1, W_l2, b_l2, W_r2):
    src2 = g[0].reshape(NW * NCH, CH)
    dst2 = g[1].reshape(NW * NCH, CH)
    x = jnp.pad(embeds, ((0, NPAD - N), (0, 0)))
    zrows1 = jnp.zeros((NPAD, F1), jnp.float32)
    zrows2 = jnp.zeros((NPAD, F2), jnp.float32)
    zn = jnp.zeros((NPAD,), jnp.float32)
    ones_c = jnp.ones((CH,), jnp.float32)

    s1, cnt = _make_seg_sum(F1, True)(src2, dst2, x, zrows1, zn, ones_c)
    cntr = cnt.reshape(NC, NPAD, 1)

    a1 = W_l1.T
    r1 = W_r1.T
    c2 = jnp.pad(W_l2, ((0, F2 - W_l2.shape[0]), (0, 0))).T
    b1r = b_l1.reshape(1, F1)
    h, y2 = _tc_layer1(s1, cntr, x, a1, b1r, r1, c2)

    s2 = _make_seg_sum(F2, False)(src2, dst2, y2, zrows2)

    r2 = jnp.pad(W_r2, ((0, F2 - W_r2.shape[0]), (0, 0))).T
    b2r = jnp.pad(b_l2, (0, F2 - b_l2.shape[0])).reshape(1, F2)
    o = _tc_layer2(s2, cntr, h, r2, b2r)
    return o[:N, :40]


# speedup vs baseline: 6.9115x; 6.9115x over previous
import functools

import jax
import jax.numpy as jnp
from jax import lax
from jax.experimental import pallas as pl
from jax.experimental.pallas import tpu as pltpu
from jax.experimental.pallas import tpu_sc as plsc

N = 10000
E = 320000
NPAD = 10240
F1 = 128
F2 = 64
NC = 2
NS = 16
NW = NC * NS
EW = E // NW
CH = 80
NCH = EW // CH
NR = NPAD // NS


def _seg_sum_body(D, with_cnt, *refs):
    if with_cnt:
        (src2, dst2, x, zrows, zn, ones_c, s_out, cnt_out,
         src_v, dst_v, rows_v, ones_v, acc_sh, cnt_sh, sem) = refs
    else:
        (src2, dst2, x, zrows, s_out,
         src_v, dst_v, rows_v, acc_sh, sem) = refs
    cid = lax.axis_index("c")
    sid = lax.axis_index("s")
    wid = cid * NS + sid

    pltpu.sync_copy(zrows.at[pl.ds(sid * NR, NR)],
                    acc_sh.at[pl.ds(sid * NR, NR)])
    if with_cnt:
        pltpu.sync_copy(zn.at[pl.ds(sid * NR, NR)],
                        cnt_sh.at[pl.ds(sid * NR, NR)])
        pltpu.sync_copy(ones_c, ones_v)
    pltpu.sync_copy(src2.at[wid], src_v)
    pltpu.sync_copy(dst2.at[wid], dst_v)
    plsc.subcore_barrier()

    def chunk(j, carry):
        pltpu.async_copy(x.at[src_v.at[j]], rows_v, sem).wait()
        pltpu.sync_copy(rows_v, acc_sh.at[dst_v.at[j]], add=True)
        if with_cnt:
            pltpu.sync_copy(ones_v, cnt_sh.at[dst_v.at[j]], add=True)
        return carry

    lax.fori_loop(0, NCH, chunk, 0)
    plsc.subcore_barrier()

    pltpu.sync_copy(acc_sh.at[pl.ds(sid * NR, NR)],
                    s_out.at[cid, pl.ds(sid * NR, NR)])
    if with_cnt:
        pltpu.sync_copy(cnt_sh.at[pl.ds(sid * NR, NR)],
                        cnt_out.at[cid, pl.ds(sid * NR, NR)])


def _make_seg_sum(D, with_cnt):
    mesh = plsc.VectorSubcoreMesh(core_axis_name="c", subcore_axis_name="s",
                                  num_cores=NC, num_subcores=NS)
    out_type = [jax.ShapeDtypeStruct((NC, NPAD, D), jnp.float32)]
    scratch = [
        pltpu.VMEM((NCH, CH), jnp.int32),
        pltpu.VMEM((NCH, CH), jnp.int32),
        pltpu.VMEM((CH, D), jnp.float32),
    ]
    if with_cnt:
        out_type.append(jax.ShapeDtypeStruct((NC, NPAD), jnp.float32))
        scratch.append(pltpu.VMEM((CH,), jnp.float32))
    scratch.append(pltpu.VMEM_SHARED((NPAD, D), jnp.float32))
    if with_cnt:
        scratch.append(pltpu.VMEM_SHARED((NPAD,), jnp.float32))
    scratch.append(pltpu.SemaphoreType.DMA)
    return pl.kernel(functools.partial(_seg_sum_body, D, with_cnt),
                     out_type=out_type, mesh=mesh, scratch_types=scratch)


def _layer1_body(s_ref, c_ref, x_ref, a1_ref, b1_ref, r1_ref, h_ref):
    s = s_ref[0] + s_ref[1]
    cnt = c_ref[0] + c_ref[1]
    mean = s / jnp.maximum(cnt, 1.0)
    h = (jnp.dot(mean, a1_ref[...], preferred_element_type=jnp.float32)
         + b1_ref[...]
         + jnp.dot(x_ref[...], r1_ref[...], preferred_element_type=jnp.float32))
    h_ref[...] = h


def _layer2_body(s_ref, c_ref, h_ref, c2_ref, r2_ref, b2_ref, o_ref):
    s = s_ref[0] + s_ref[1]
    cnt = c_ref[0] + c_ref[1]
    mean = s / jnp.maximum(cnt, 1.0)
    o_ref[...] = (jnp.dot(mean, c2_ref[...], preferred_element_type=jnp.float32)
                  + b2_ref[...]
                  + jnp.dot(h_ref[...], r2_ref[...],
                            preferred_element_type=jnp.float32))


BR = 1024
GRID = NPAD // BR


def _tc_layer1(s1, cntr, x, a1, b1, r1):
    return pl.pallas_call(
        _layer1_body,
        grid=(GRID,),
        in_specs=[
            pl.BlockSpec((NC, BR, F1), lambda i: (0, i, 0)),
            pl.BlockSpec((NC, BR, 1), lambda i: (0, i, 0)),
            pl.BlockSpec((BR, F1), lambda i: (i, 0)),
            pl.BlockSpec((F1, F1), lambda i: (0, 0)),
            pl.BlockSpec((1, F1), lambda i: (0, 0)),
            pl.BlockSpec((F1, F1), lambda i: (0, 0)),
        ],
        out_specs=pl.BlockSpec((BR, F1), lambda i: (i, 0)),
        out_shape=jax.ShapeDtypeStruct((NPAD, F1), jnp.float32),
    )(s1, cntr, x, a1, b1, r1)


def _tc_layer2(s2, cntr, h, c2, r2, b2):
    return pl.pallas_call(
        _layer2_body,
        grid=(GRID,),
        in_specs=[
            pl.BlockSpec((NC, BR, F1), lambda i: (0, i, 0)),
            pl.BlockSpec((NC, BR, 1), lambda i: (0, i, 0)),
            pl.BlockSpec((BR, F1), lambda i: (i, 0)),
            pl.BlockSpec((F1, F2), lambda i: (0, 0)),
            pl.BlockSpec((F1, F2), lambda i: (0, 0)),
            pl.BlockSpec((1, F2), lambda i: (0, 0)),
        ],
        out_specs=pl.BlockSpec((BR, F2), lambda i: (i, 0)),
        out_shape=jax.ShapeDtypeStruct((NPAD, F2), jnp.float32),
    )(s2, cntr, h, c2, r2, b2)


def kernel(g, embeds, W_l1, b_l1, W_r1, W_l2, b_l2, W_r2):
    src2 = g[0].reshape(NW, NCH, CH)
    dst2 = g[1].reshape(NW, NCH, CH)
    x = jnp.pad(embeds, ((0, NPAD - N), (0, 0)))
    zrows1 = jnp.zeros((NPAD, F1), jnp.float32)
    zn = jnp.zeros((NPAD,), jnp.float32)
    ones_c = jnp.ones((CH,), jnp.float32)

    s1, cnt = _make_seg_sum(F1, True)(src2, dst2, x, zrows1, zn, ones_c)
    cntr = cnt.reshape(NC, NPAD, 1)

    a1 = W_l1.T
    r1 = W_r1.T
    b1r = b_l1.reshape(1, F1)
    h = _tc_layer1(s1, cntr, x, a1, b1r, r1)

    (s2,) = _make_seg_sum(F1, False)(src2, dst2, h, zrows1)

    c2 = jnp.pad(W_l2, ((0, F2 - W_l2.shape[0]), (0, 0))).T
    r2 = jnp.pad(W_r2, ((0, F2 - W_r2.shape[0]), (0, 0))).T
    b2r = jnp.pad(b_l2, (0, F2 - b_l2.shape[0])).reshape(1, F2)
    o = _tc_layer2(s2, cntr, h, c2, r2, b2r)
    return o[:N, :40]
